# baseline (speedup 1.0000x reference)
import functools

import jax
import jax.numpy as jnp
import numpy as np
from jax import lax
from jax.experimental import pallas as pl
from jax.experimental.pallas import tpu as pltpu
from jax.experimental.pallas import tpu_sc as plsc

_N, _T, _C, _H, _W = 8, 16, 3, 224, 224
_B = _N * _T
_CX = _CY = 112

_OX = np.array([49, 36, 194, 131, 25, 219, 30, 129, 143, 11, 76, 111, 57, 102, 91, 214, 44, 22, 158, 49, 173, 170, 28, 13, 150, 187, 22, 140, 42, 144, 66, 112, 195, 107, 64, 46, 175, 33, 150, 31, 209, 92, 29, 213, 209, 11, 87, 218, 94, 216, 128, 160, 45, 21, 197, 57, 129, 94, 124, 153, 180, 208, 207, 102, 219, 108, 15, 192, 29, 176, 76, 159, 86, 47, 70, 177, 172, 126, 43, 102, 211, 223, 25, 179, 202, 89, 169, 204, 90, 79, 188, 54, 35, 110, 167, 139, 30, 186, 158, 39, 217, 41, 148, 175, 119, 28, 120, 207, 91, 69, 195, 155, 170, 96, 1, 38, 56, 142, 5, 26, 187, 18, 40, 20, 59, 212, 73, 87], dtype=np.int32)
_OY = np.array([110, 122, 4, 122, 160, 129, 166, 13, 56, 175, 200, 36, 213, 201, 93, 78, 156, 7, 220, 217, 194, 108, 91, 20, 168, 149, 148, 128, 206, 93, 63, 190, 195, 213, 97, 181, 31, 165, 182, 185, 154, 154, 102, 130, 103, 169, 129, 150, 170, 92, 8, 110, 3, 174, 151, 80, 138, 51, 49, 200, 78, 35, 105, 216, 153, 77, 164, 4, 48, 22, 15, 137, 191, 154, 33, 80, 33, 72, 6, 144, 110, 108, 57, 149, 66, 56, 107, 58, 138, 50, 95, 58, 151, 107, 95, 61, 55, 81, 30, 8, 82, 169, 128, 99, 63, 149, 65, 222, 15, 166, 126, 62, 204, 38, 109, 145, 149, 93, 47, 204, 88, 219, 116, 135, 80, 47, 32, 148], dtype=np.int32)

_R0 = np.clip(_OX - _CX // 2, 0, _H - 1)
_R1 = np.clip(_OX - _CX // 2 + _CX - 1, 0, _H - 1)
_C0 = np.clip(_OY - _CY // 2, 0, _W - 1)
_C1 = np.clip(_OY - _CY // 2 + _CY - 1, 0, _W - 1)
_RECTS = np.stack([_R0, _R1, _C0, _C1], axis=1).astype(np.int32)


_NW = 32
_HALF = _H // 2
_CHUNK = _HALF * _W


def _sc_table(lo: int, hi: int, per_w: int) -> np.ndarray:
    tbl = np.zeros((_NW, per_w, 16), dtype=np.int32)
    for cid in range((hi - lo) * 2 * _C):
        b = lo + cid // (2 * _C)
        half = cid % 2
        rowbase = half * _HALF
        rlo_g = max(int(_R0[b]), rowbase)
        rhi_g = min(int(_R1[b]), rowbase + _HALF - 1)
        if rhi_g < rlo_g:
            row = [0, 0, 0, 0, 0]
        else:
            c0, c1 = int(_C0[b]), int(_C1[b])
            row = [rlo_g - rowbase, rhi_g - rowbase + 1, c0, c1, c0 // 16]
        w, i = divmod(cid, per_w)
        tbl[w, i, : len(row)] = row
    return tbl


def _make_sc_erase(lo: int, hi: int):
    nchunk = (hi - lo) * 2 * _C
    per_w = nchunk // _NW
    nbuf = min(5, per_w)
    dist = min(3, per_w)
    tbl_np = _sc_table(lo, hi, per_w)

    def body(f_hbm, tbl_hbm, out_hbm, tbl_v, *scratch):
        bufs = scratch[:nbuf]
        isems = scratch[nbuf : 2 * nbuf]
        osems = scratch[2 * nbuf : 3 * nbuf]
        wid = lax.axis_index("s") * 2 + lax.axis_index("c")
        pltpu.sync_copy(tbl_hbm.at[wid], tbl_v)
        base = wid * per_w

        lanes = lax.broadcasted_iota(jnp.int32, (16,), 0)
        in_h, out_h = {}, {}

        def start_in(i):
            j = i % nbuf
            h = pltpu.make_async_copy(
                f_hbm.at[pl.ds((base + i) * _CHUNK, _CHUNK)], bufs[j], isems[j])
            h.start()
            in_h[i] = h

        def start_out(i):
            j = i % nbuf
            h = pltpu.make_async_copy(
                bufs[j], out_hbm.at[pl.ds((base + i) * _CHUNK, _CHUNK)], osems[j])
            h.start()
            out_h[i] = h

        for i in range(dist):
            start_in(i)

        for i in range(per_w):
            j = i % nbuf
            in_h[i].wait()
            v = tbl_v[i]
            rlo, rhi, c0, c1, g0 = v[0], v[1], v[2], v[3], v[4]

            offs, msks = [], []
            for jj in range(8):
                gbase0 = (g0 + jj) * 16
                gbase = jnp.where(gbase0 > _W - 16, gbase0 - _W, gbase0)
                col = gbase + lanes
                offs.append(gbase)
                msks.append((col >= c0) & (col <= c1))

            @plsc.parallel_loop(rlo, rhi, unroll=2)
            def _(r, j=j):
                rowbase = r * _W
                for jj in range(8):
                    start = rowbase + offs[jj]
                    cur = bufs[j][pl.ds(start, 16)]
                    bufs[j][pl.ds(start, 16)] = jnp.where(msks[jj], 0.0, cur)

            start_out(i)
            k = i + dist
            if k < per_w:
                if k >= nbuf:
                    out_h[k - nbuf].wait()
                start_in(k)

        for i in range(max(0, per_w - nbuf), per_w):
            out_h[i].wait()

    sc_call = functools.partial(
        pl.kernel,
        mesh=plsc.VectorSubcoreMesh(core_axis_name="c", subcore_axis_name="s"),
        out_type=jax.ShapeDtypeStruct((nchunk * _CHUNK,), jnp.float32),
        scratch_types=(
            [pltpu.VMEM((per_w, 16), jnp.int32)]
            + [pltpu.VMEM((_CHUNK,), jnp.float32) for _ in range(nbuf)]
            + [pltpu.SemaphoreType.DMA for _ in range(2 * nbuf)]
        ),
    )(body)

    def run(fpart_flat):
        return sc_call(fpart_flat, jnp.asarray(tbl_np))

    return run


_BLOCK_B = 16


def _make_tc_erase(lo: int, hi: int):
    nb = hi - lo
    rects_np = _RECTS[lo:hi]

    def erase_body(rect_ref, x_ref, o_ref):
        g = pl.program_id(0)
        rows = jax.lax.broadcasted_iota(jnp.int32, (_H, _W), 0)
        cols = jax.lax.broadcasted_iota(jnp.int32, (_H, _W), 1)
        for i in range(_BLOCK_B):
            b = g * _BLOCK_B + i
            r0 = rect_ref[b, 0]
            r1 = rect_ref[b, 1]
            c0 = rect_ref[b, 2]
            c1 = rect_ref[b, 3]
            inside = (rows >= r0) & (rows <= r1) & (cols >= c0) & (cols <= c1)
            o_ref[i] = jnp.where(inside[None, :, :], 0.0, x_ref[i])

    def run(fpart):
        return pl.pallas_call(
            erase_body,
            grid=(nb // _BLOCK_B,),
            in_specs=[
                pl.BlockSpec(memory_space=pltpu.SMEM),
                pl.BlockSpec((_BLOCK_B, _C, _H, _W), lambda b: (b, 0, 0, 0)),
            ],
            out_specs=pl.BlockSpec((_BLOCK_B, _C, _H, _W), lambda b: (b, 0, 0, 0)),
            out_shape=jax.ShapeDtypeStruct(fpart.shape, fpart.dtype),
        )(jnp.asarray(rects_np), fpart)

    return run


def _make_tc_erase_flat(lo: int, hi: int):
    nb = hi - lo
    rects_np = _RECTS[lo:hi]
    rows_per_frame = _C * _H * _W // 128

    def erase_body(rect_ref, x_ref, o_ref):
        g = pl.program_id(0)
        ridx = jax.lax.broadcasted_iota(jnp.int32, (rows_per_frame, 128), 0)
        lidx = jax.lax.broadcasted_iota(jnp.int32, (rows_per_frame, 128), 1)
        idx = ridx * 128 + lidx
        rem = idx % (_H * _W)
        row = rem // _W
        col = rem - row * _W
        for i in range(_BLOCK_B):
            b = g * _BLOCK_B + i
            r0 = rect_ref[b, 0]
            r1 = rect_ref[b, 1]
            c0 = rect_ref[b, 2]
            c1 = rect_ref[b, 3]
            inside = (row >= r0) & (row <= r1) & (col >= c0) & (col <= c1)
            o_ref[i] = jnp.where(inside, 0.0, x_ref[i])

    def run(fpart_flat3):
        return pl.pallas_call(
            erase_body,
            grid=(nb // _BLOCK_B,),
            in_specs=[
                pl.BlockSpec(memory_space=pltpu.SMEM),
                pl.BlockSpec((_BLOCK_B, rows_per_frame, 128), lambda b: (b, 0, 0)),
            ],
            out_specs=pl.BlockSpec((_BLOCK_B, rows_per_frame, 128), lambda b: (b, 0, 0)),
            out_shape=jax.ShapeDtypeStruct(fpart_flat3.shape, fpart_flat3.dtype),
        )(jnp.asarray(rects_np), fpart_flat3)

    return run


_SPLIT = 128

_tc_part = _make_tc_erase_flat(0, _SPLIT) if _SPLIT > 0 else None
_sc_part = _make_sc_erase(_SPLIT, _B) if _SPLIT < _B else None


@jax.jit
def kernel(frames):
    n, t, c, h, w = frames.shape
    f = frames.reshape(_B, c, h, w)
    parts = []
    if _sc_part is not None:
        sc_out = _sc_part(f[_SPLIT:].reshape(-1)).reshape(_B - _SPLIT, c, h, w)
    if _tc_part is not None:
        tc_out = _tc_part(f[:_SPLIT].reshape(_SPLIT, _C * _H * _W // 128, 128))
        parts.append(tc_out.reshape(_SPLIT, c, h, w))
    if _sc_part is not None:
        parts.append(sc_out)
    out = parts[0] if len(parts) == 1 else jnp.concatenate(parts, axis=0)
    return out.reshape(n, t, c, h, w)


# speedup vs baseline: 4.5853x; 4.5853x over previous
import functools

import jax
import jax.numpy as jnp
import numpy as np
from jax import lax
from jax.experimental import pallas as pl
from jax.experimental.pallas import tpu as pltpu
from jax.experimental.pallas import tpu_sc as plsc

_N, _T, _C, _H, _W = 8, 16, 3, 224, 224
_B = _N * _T
_CX = _CY = 112

_OX = np.array([49, 36, 194, 131, 25, 219, 30, 129, 143, 11, 76, 111, 57, 102, 91, 214, 44, 22, 158, 49, 173, 170, 28, 13, 150, 187, 22, 140, 42, 144, 66, 112, 195, 107, 64, 46, 175, 33, 150, 31, 209, 92, 29, 213, 209, 11, 87, 218, 94, 216, 128, 160, 45, 21, 197, 57, 129, 94, 124, 153, 180, 208, 207, 102, 219, 108, 15, 192, 29, 176, 76, 159, 86, 47, 70, 177, 172, 126, 43, 102, 211, 223, 25, 179, 202, 89, 169, 204, 90, 79, 188, 54, 35, 110, 167, 139, 30, 186, 158, 39, 217, 41, 148, 175, 119, 28, 120, 207, 91, 69, 195, 155, 170, 96, 1, 38, 56, 142, 5, 26, 187, 18, 40, 20, 59, 212, 73, 87], dtype=np.int32)
_OY = np.array([110, 122, 4, 122, 160, 129, 166, 13, 56, 175, 200, 36, 213, 201, 93, 78, 156, 7, 220, 217, 194, 108, 91, 20, 168, 149, 148, 128, 206, 93, 63, 190, 195, 213, 97, 181, 31, 165, 182, 185, 154, 154, 102, 130, 103, 169, 129, 150, 170, 92, 8, 110, 3, 174, 151, 80, 138, 51, 49, 200, 78, 35, 105, 216, 153, 77, 164, 4, 48, 22, 15, 137, 191, 154, 33, 80, 33, 72, 6, 144, 110, 108, 57, 149, 66, 56, 107, 58, 138, 50, 95, 58, 151, 107, 95, 61, 55, 81, 30, 8, 82, 169, 128, 99, 63, 149, 65, 222, 15, 166, 126, 62, 204, 38, 109, 145, 149, 93, 47, 204, 88, 219, 116, 135, 80, 47, 32, 148], dtype=np.int32)

_R0 = np.clip(_OX - _CX // 2, 0, _H - 1)
_R1 = np.clip(_OX - _CX // 2 + _CX - 1, 0, _H - 1)
_C0 = np.clip(_OY - _CY // 2, 0, _W - 1)
_C1 = np.clip(_OY - _CY // 2 + _CY - 1, 0, _W - 1)
_RECTS = np.stack([_R0, _R1, _C0, _C1], axis=1).astype(np.int32)


_NW = 32
_HALF = _H // 2
_CHUNK = _HALF * _W


def _sc_table(lo: int, hi: int, per_w: int) -> np.ndarray:
    tbl = np.zeros((_NW, per_w, 16), dtype=np.int32)
    for cid in range((hi - lo) * 2 * _C):
        b = lo + cid // (2 * _C)
        half = cid % 2
        rowbase = half * _HALF
        rlo_g = max(int(_R0[b]), rowbase)
        rhi_g = min(int(_R1[b]), rowbase + _HALF - 1)
        if rhi_g < rlo_g:
            row = [0, 0, 0, 0, 0]
        else:
            c0, c1 = int(_C0[b]), int(_C1[b])
            row = [rlo_g - rowbase, rhi_g - rowbase + 1, c0, c1, c0 // 16]
        w, i = divmod(cid, per_w)
        tbl[w, i, : len(row)] = row
    return tbl


def _make_sc_erase(lo: int, hi: int):
    nchunk = (hi - lo) * 2 * _C
    per_w = nchunk // _NW
    nbuf = min(5, per_w)
    dist = min(3, per_w)
    tbl_np = _sc_table(lo, hi, per_w)

    def body(f_hbm, tbl_hbm, out_hbm, tbl_v, *scratch):
        bufs = scratch[:nbuf]
        isems = scratch[nbuf : 2 * nbuf]
        osems = scratch[2 * nbuf : 3 * nbuf]
        wid = lax.axis_index("s") * 2 + lax.axis_index("c")
        pltpu.sync_copy(tbl_hbm.at[wid], tbl_v)
        base = wid * per_w

        lanes = lax.broadcasted_iota(jnp.int32, (16,), 0)
        in_h, out_h = {}, {}

        def start_in(i):
            j = i % nbuf
            h = pltpu.make_async_copy(
                f_hbm.at[pl.ds((base + i) * _CHUNK, _CHUNK)], bufs[j], isems[j])
            h.start()
            in_h[i] = h

        def start_out(i):
            j = i % nbuf
            h = pltpu.make_async_copy(
                bufs[j], out_hbm.at[pl.ds((base + i) * _CHUNK, _CHUNK)], osems[j])
            h.start()
            out_h[i] = h

        for i in range(dist):
            start_in(i)

        for i in range(per_w):
            j = i % nbuf
            in_h[i].wait()
            v = tbl_v[i]
            rlo, rhi, c0, c1, g0 = v[0], v[1], v[2], v[3], v[4]

            offs, msks = [], []
            for jj in range(8):
                gbase0 = (g0 + jj) * 16
                gbase = jnp.where(gbase0 > _W - 16, gbase0 - _W, gbase0)
                col = gbase + lanes
                offs.append(gbase)
                msks.append((col >= c0) & (col <= c1))

            @plsc.parallel_loop(rlo, rhi, unroll=2)
            def _(r, j=j):
                rowbase = r * _W
                for jj in range(8):
                    start = rowbase + offs[jj]
                    cur = bufs[j][pl.ds(start, 16)]
                    bufs[j][pl.ds(start, 16)] = jnp.where(msks[jj], 0.0, cur)

            start_out(i)
            k = i + dist
            if k < per_w:
                if k >= nbuf:
                    out_h[k - nbuf].wait()
                start_in(k)

        for i in range(max(0, per_w - nbuf), per_w):
            out_h[i].wait()

    sc_call = functools.partial(
        pl.kernel,
        mesh=plsc.VectorSubcoreMesh(core_axis_name="c", subcore_axis_name="s"),
        out_type=jax.ShapeDtypeStruct((nchunk * _CHUNK,), jnp.float32),
        scratch_types=(
            [pltpu.VMEM((per_w, 16), jnp.int32)]
            + [pltpu.VMEM((_CHUNK,), jnp.float32) for _ in range(nbuf)]
            + [pltpu.SemaphoreType.DMA for _ in range(2 * nbuf)]
        ),
    )(body)

    def run(fpart_flat):
        return sc_call(fpart_flat, jnp.asarray(tbl_np))

    return run


_BLOCK_B = 16


def _make_tc_erase(lo: int, hi: int):
    nb = hi - lo
    rects_np = _RECTS[lo:hi]

    def erase_body(rect_ref, x_ref, o_ref):
        g = pl.program_id(0)
        rows = jax.lax.broadcasted_iota(jnp.int32, (_H, _W), 0)
        cols = jax.lax.broadcasted_iota(jnp.int32, (_H, _W), 1)
        for i in range(_BLOCK_B):
            b = g * _BLOCK_B + i
            r0 = rect_ref[b, 0]
            r1 = rect_ref[b, 1]
            c0 = rect_ref[b, 2]
            c1 = rect_ref[b, 3]
            inside = (rows >= r0) & (rows <= r1) & (cols >= c0) & (cols <= c1)
            o_ref[i] = jnp.where(inside[None, :, :], 0.0, x_ref[i])

    def run(fpart):
        return pl.pallas_call(
            erase_body,
            grid=(nb // _BLOCK_B,),
            in_specs=[
                pl.BlockSpec(memory_space=pltpu.SMEM),
                pl.BlockSpec((_BLOCK_B, _C, _H, _W), lambda b: (b, 0, 0, 0)),
            ],
            out_specs=pl.BlockSpec((_BLOCK_B, _C, _H, _W), lambda b: (b, 0, 0, 0)),
            out_shape=jax.ShapeDtypeStruct(fpart.shape, fpart.dtype),
        )(jnp.asarray(rects_np), fpart)

    return run


_SPLIT = 128

_tc_part = _make_tc_erase(0, _SPLIT) if _SPLIT > 0 else None
_sc_part = _make_sc_erase(_SPLIT, _B) if _SPLIT < _B else None


@jax.jit
def kernel(frames):
    n, t, c, h, w = frames.shape
    f = frames.reshape(_B, c, h, w)
    parts = []
    if _sc_part is not None:
        sc_out = _sc_part(f[_SPLIT:].reshape(-1)).reshape(_B - _SPLIT, c, h, w)
    if _tc_part is not None:
        parts.append(_tc_part(f[:_SPLIT]))
    if _sc_part is not None:
        parts.append(sc_out)
    out = parts[0] if len(parts) == 1 else jnp.concatenate(parts, axis=0)
    return out.reshape(n, t, c, h, w)
